# baseline (speedup 1.0000x reference)
import functools

import jax
import jax.numpy as jnp
from jax import lax
from jax.experimental import pallas as pl
from jax.experimental.pallas import tpu as pltpu
from jax.experimental.pallas import tpu_sc as plsc

NEG_SLOPE = 0.2
_NC, _NS, _L = 2, 16, 16
_CH = 48


def _proj_body(x_ref, wf_ref, bf_ref, wr_ref, br_ref, f_ref, er_ref):
    x = x_ref[...]
    hi = jax.lax.Precision.HIGHEST
    f_ref[...] = jnp.dot(x, wf_ref[...], precision=hi,
                         preferred_element_type=jnp.float32) + bf_ref[...]
    er_ref[...] = jnp.dot(x, wr_ref[...], precision=hi,
                          preferred_element_type=jnp.float32) + br_ref[...]


def _edge_body(n_pad, k, d_out, nchunk,
               f_hbm, er_hbm, idx_hbm, z_hbm,
               acc_out,
               acc, idx_i,
               sg0, sg1, dg0, dg1, ds0, ds1,
               r_v0, r_v1, f_v0, f_v1, o_v0, o_v1,
               gs0, gs1, ss0, ss1):
    cid = lax.axis_index("c")
    sid = lax.axis_index("s")
    wid = sid * _NC + cid
    kd = k * d_out
    w = kd + _L

    src_g = (sg0, sg1)
    dst_g = (dg0, dg1)
    dst_s = (ds0, ds1)
    r_v = (r_v0, r_v1)
    f_v = (f_v0, f_v1)
    o_v = (o_v0, o_v1)
    gsem = (gs0, gs1)
    ssem = (ss0, ss1)

    rows = n_pad // _NS
    r0 = sid * rows
    pltpu.sync_copy(z_hbm.at[pl.ds(r0, rows)], acc.at[pl.ds(r0, rows)])
    pltpu.sync_copy(idx_hbm.at[pl.ds(wid * nchunk, nchunk)], idx_i)
    plsc.subcore_barrier()

    def unpack(c, sbuf, dbuf):
        for j in range(_CH // _L):
            wv = idx_i[c, pl.ds(j * _L, _L)]
            if sbuf is not None:
                sbuf[pl.ds(j * _L, _L)] = jnp.bitwise_and(wv, 0xFFFF)
            dbuf[pl.ds(j * _L, _L)] = lax.shift_right_logical(wv, 16)

    def issue_gathers(b):
        pltpu.async_copy(f_hbm.at[src_g[b]], f_v[b], gsem[b])
        pltpu.async_copy(er_hbm.at[dst_g[b]], r_v[b], gsem[b])

    def wait_gathers(b):
        pltpu.make_async_copy(f_hbm.at[pl.ds(0, _CH)], f_v[b], gsem[b]).wait()
        pltpu.make_async_copy(er_hbm.at[pl.ds(0, _CH)], r_v[b], gsem[b]).wait()

    def wait_scatter(b):
        pltpu.make_async_copy(f_hbm.at[pl.ds(0, _CH)], o_v[b], ssem[b]).wait()

    def compute(c, b, first):
        wait_gathers(b)

        @pl.when(c + 2 < nchunk)
        def _():
            unpack(c + 2, src_g[b], dst_g[b])

        def ebody(j, carry):
            v = f_v[b][j, pl.ds(kd, _L)] + r_v[b][j, :]
            v = jnp.where(v > 0.0, v, NEG_SLOPE * v)
            sv = jnp.exp(v)
            o_v[b][j, pl.ds(kd, _L)] = sv
            for kk in range(k):
                sl = pl.ds(kk * d_out, d_out)
                o_v[b][j, sl] = f_v[b][j, sl] * sv[kk]
            return carry

        lax.fori_loop(0, _CH, ebody, 0, unroll=4)
        if not first:
            wait_scatter(b)
        unpack(c, None, dst_s[b])
        pltpu.async_copy(o_v[b], acc.at[dst_s[b]], ssem[b], add=True)

        @pl.when(c + 2 < nchunk)
        def _():
            issue_gathers(b)

    unpack(0, src_g[0], dst_g[0])
    issue_gathers(0)
    unpack(1, src_g[1], dst_g[1])
    issue_gathers(1)
    compute(0, 0, True)
    compute(1, 1, True)

    def pair_body(i, carry):
        compute(2 * i, 0, False)
        compute(2 * i + 1, 1, False)
        return carry

    lax.fori_loop(1, nchunk // 2, pair_body, 0)
    wait_scatter(0)
    wait_scatter(1)
    plsc.subcore_barrier()

    pltpu.sync_copy(acc.at[pl.ds(r0, rows)], acc_out.at[cid, pl.ds(r0, rows)])


def _combine_body(kd, acc_ref, em_ref, o_ref):
    a = acc_ref[0] + acc_ref[1]
    d = a[:, kd:]
    dexp = jnp.dot(d, em_ref[...], precision=jax.lax.Precision.HIGHEST,
                   preferred_element_type=jnp.float32)
    o_ref[...] = jnp.where(dexp > 0.0, a[:, :kd] / dexp, 0.0)


def kernel(feat, edge_index, W_src, b_src, W_dst, b_dst, attn_src):
    n, d_in = feat.shape
    e = edge_index.shape[1]
    k = attn_src.shape[0]
    d_out = attn_src.shape[1] // 2
    kd = k * d_out
    w = kd + _L
    f32 = jnp.float32

    nw = _NC * _NS
    assert d_out == _L and n < 0x8000
    nchunk = (((e + nw * _CH - 1) // (nw * _CH)) + 1) // 2 * 2
    e_pad = nw * nchunk * _CH
    n_pad = ((n + 1 + 8 * _NS - 1) // (8 * _NS)) * (8 * _NS)

    attn_l = attn_src[:, :d_out]
    attn_r = attn_src[:, d_out:]
    w_el = jnp.einsum('dkc,kc->dk', W_src.reshape(d_in, k, d_out), attn_l)
    b_el = jnp.einsum('kc,kc->k', b_src.reshape(k, d_out), attn_l)
    w_er = jnp.einsum('dkc,kc->dk', W_dst.reshape(d_in, k, d_out), attn_r)
    b_er = jnp.einsum('kc,kc->k', b_dst.reshape(k, d_out), attn_r)
    pad = _L - k
    w_f = jnp.concatenate([W_src, jnp.pad(w_el, ((0, 0), (0, pad)))], axis=1)
    b_f = jnp.concatenate([b_src, jnp.pad(b_el, ((0, pad),))])
    w_er_p = jnp.pad(w_er, ((0, 0), (0, pad)))
    b_er_p = jnp.pad(b_er, ((0, pad),))

    fs, erp = pl.pallas_call(
        _proj_body,
        out_shape=(
            jax.ShapeDtypeStruct((n, w), f32),
            jax.ShapeDtypeStruct((n, _L), f32),
        ),
    )(feat, w_f, b_f, w_er_p, b_er_p)

    packed = jnp.concatenate([
        edge_index[0] + (edge_index[1] << 16),
        jnp.full((e_pad - e,), n << 16, jnp.int32),
    ]).reshape(nw * nchunk, _CH)
    z = jnp.zeros((n_pad, w), f32)

    ib = pltpu.VMEM((_CH,), jnp.int32)
    ch_l = pltpu.VMEM((_CH, _L), f32)
    ch_f = pltpu.VMEM((_CH, w), f32)
    sem = pltpu.SemaphoreType.DMA
    sc_edge = pl.kernel(
        functools.partial(_edge_body, n_pad, k, d_out, nchunk),
        out_type=jax.ShapeDtypeStruct((_NC, n_pad, w), f32),
        mesh=plsc.VectorSubcoreMesh(core_axis_name="c", subcore_axis_name="s"),
        scratch_types=[
            pltpu.VMEM_SHARED((n_pad, w), f32),
            pltpu.VMEM((nchunk, _CH), jnp.int32),
            ib, ib, ib, ib, ib, ib,
            ch_l, ch_l,
            ch_f, ch_f,
            ch_f, ch_f,
            sem, sem, sem, sem,
        ],
        compiler_params=pltpu.CompilerParams(use_tc_tiling_on_sc=False),
    )
    acc_p = sc_edge(fs, erp, packed, z)

    emat = (jnp.arange(kd)[None, :] // d_out ==
            jnp.arange(_L)[:, None]).astype(f32)
    out = pl.pallas_call(
        functools.partial(_combine_body, kd),
        out_shape=jax.ShapeDtypeStruct((n_pad, kd), f32),
    )(acc_p, emat)
    return out[:n]


# speedup vs baseline: 1.7268x; 1.7268x over previous
import functools

import jax
import jax.numpy as jnp
from jax import lax
from jax.experimental import pallas as pl
from jax.experimental.pallas import tpu as pltpu
from jax.experimental.pallas import tpu_sc as plsc

NEG_SLOPE = 0.2
_NC, _NS, _L = 2, 16, 16
_CH = 48


def _proj_body(x_ref, wf_ref, bf_ref, wr_ref, br_ref, f_ref, er_ref):
    x = x_ref[...]
    hi = jax.lax.Precision.HIGHEST
    f_ref[...] = jnp.dot(x, wf_ref[...], precision=hi,
                         preferred_element_type=jnp.float32) + bf_ref[...]
    er_ref[...] = jnp.dot(x, wr_ref[...], precision=hi,
                          preferred_element_type=jnp.float32) + br_ref[...]


def _edge_body(n_pad, k, d_out, nchunk,
               f_hbm, er_hbm, idx_hbm, z_hbm,
               acc_out,
               acc, idx_i,
               sg0, sg1, dg0, dg1, ds0, ds1,
               r_v0, r_v1, f_v0, f_v1, o_v0, o_v1,
               gs0, gs1, ss0, ss1):
    cid = lax.axis_index("c")
    sid = lax.axis_index("s")
    wid = sid * _NC + cid
    kd = k * d_out
    w = kd + _L

    src_g = (sg0, sg1)
    dst_g = (dg0, dg1)
    dst_s = (ds0, ds1)
    r_v = (r_v0, r_v1)
    f_v = (f_v0, f_v1)
    o_v = (o_v0, o_v1)
    gsem = (gs0, gs1)
    ssem = (ss0, ss1)

    rows = n_pad // _NS
    r0 = sid * rows
    pltpu.sync_copy(z_hbm.at[pl.ds(r0, rows)], acc.at[pl.ds(r0, rows)])
    pltpu.sync_copy(idx_hbm.at[pl.ds(wid * nchunk, nchunk)], idx_i)
    plsc.subcore_barrier()

    def unpack(c, sbuf, dbuf):
        for j in range(_CH // _L):
            wv = idx_i[c, pl.ds(j * _L, _L)]
            if sbuf is not None:
                sbuf[pl.ds(j * _L, _L)] = jnp.bitwise_and(wv, 0xFFFF)
            dbuf[pl.ds(j * _L, _L)] = lax.shift_right_logical(wv, 16)

    def issue_gathers(b):
        pltpu.async_copy(f_hbm.at[src_g[b]], f_v[b], gsem[b])
        pltpu.async_copy(er_hbm.at[dst_g[b]], r_v[b], gsem[b])

    def wait_gathers(b):
        pltpu.make_async_copy(f_hbm.at[pl.ds(0, _CH)], f_v[b], gsem[b]).wait()
        pltpu.make_async_copy(er_hbm.at[pl.ds(0, _CH)], r_v[b], gsem[b]).wait()

    def wait_scatter(b):
        pltpu.make_async_copy(f_hbm.at[pl.ds(0, _CH)], o_v[b], ssem[b]).wait()

    def compute(c, b, first):
        wait_gathers(b)

        @pl.when(c + 2 < nchunk)
        def _():
            unpack(c + 2, src_g[b], dst_g[b])

        def ebody(j):
            v = f_v[b][j, pl.ds(kd, _L)] + r_v[b][j, :]
            v = jnp.where(v > 0.0, v, NEG_SLOPE * v)
            sv = jnp.exp(v)
            o_v[b][j, pl.ds(kd, _L)] = sv
            for kk in range(k):
                sl = pl.ds(kk * d_out, d_out)
                o_v[b][j, sl] = f_v[b][j, sl] * sv[kk]

        plsc.parallel_loop(0, _CH, 1, unroll=2)(ebody)
        if not first:
            wait_scatter(b)
        unpack(c, None, dst_s[b])
        pltpu.async_copy(o_v[b], acc.at[dst_s[b]], ssem[b], add=True)

        @pl.when(c + 2 < nchunk)
        def _():
            issue_gathers(b)

    unpack(0, src_g[0], dst_g[0])
    issue_gathers(0)
    unpack(1, src_g[1], dst_g[1])
    issue_gathers(1)
    compute(0, 0, True)
    compute(1, 1, True)

    def pair_body(i, carry):
        compute(2 * i, 0, False)
        compute(2 * i + 1, 1, False)
        return carry

    lax.fori_loop(1, nchunk // 2, pair_body, 0)
    wait_scatter(0)
    wait_scatter(1)
    plsc.subcore_barrier()

    pltpu.sync_copy(acc.at[pl.ds(r0, rows)], acc_out.at[cid, pl.ds(r0, rows)])


def _combine_body(kd, acc_ref, em_ref, o_ref):
    a = acc_ref[0] + acc_ref[1]
    d = a[:, kd:]
    dexp = jnp.dot(d, em_ref[...], precision=jax.lax.Precision.HIGHEST,
                   preferred_element_type=jnp.float32)
    o_ref[...] = jnp.where(dexp > 0.0, a[:, :kd] / dexp, 0.0)


def kernel(feat, edge_index, W_src, b_src, W_dst, b_dst, attn_src):
    n, d_in = feat.shape
    e = edge_index.shape[1]
    k = attn_src.shape[0]
    d_out = attn_src.shape[1] // 2
    kd = k * d_out
    w = kd + _L
    f32 = jnp.float32

    nw = _NC * _NS
    assert d_out == _L and n < 0x8000
    nchunk = (((e + nw * _CH - 1) // (nw * _CH)) + 1) // 2 * 2
    e_pad = nw * nchunk * _CH
    n_pad = ((n + 1 + 8 * _NS - 1) // (8 * _NS)) * (8 * _NS)

    attn_l = attn_src[:, :d_out]
    attn_r = attn_src[:, d_out:]
    w_el = jnp.einsum('dkc,kc->dk', W_src.reshape(d_in, k, d_out), attn_l)
    b_el = jnp.einsum('kc,kc->k', b_src.reshape(k, d_out), attn_l)
    w_er = jnp.einsum('dkc,kc->dk', W_dst.reshape(d_in, k, d_out), attn_r)
    b_er = jnp.einsum('kc,kc->k', b_dst.reshape(k, d_out), attn_r)
    pad = _L - k
    w_f = jnp.concatenate([W_src, jnp.pad(w_el, ((0, 0), (0, pad)))], axis=1)
    b_f = jnp.concatenate([b_src, jnp.pad(b_el, ((0, pad),))])
    w_er_p = jnp.pad(w_er, ((0, 0), (0, pad)))
    b_er_p = jnp.pad(b_er, ((0, pad),))

    fs, erp = pl.pallas_call(
        _proj_body,
        out_shape=(
            jax.ShapeDtypeStruct((n, w), f32),
            jax.ShapeDtypeStruct((n, _L), f32),
        ),
    )(feat, w_f, b_f, w_er_p, b_er_p)

    packed = jnp.concatenate([
        edge_index[0] + (edge_index[1] << 16),
        jnp.full((e_pad - e,), n << 16, jnp.int32),
    ]).reshape(nw * nchunk, _CH)
    z = jnp.zeros((n_pad, w), f32)

    ib = pltpu.VMEM((_CH,), jnp.int32)
    ch_l = pltpu.VMEM((_CH, _L), f32)
    ch_f = pltpu.VMEM((_CH, w), f32)
    sem = pltpu.SemaphoreType.DMA
    sc_edge = pl.kernel(
        functools.partial(_edge_body, n_pad, k, d_out, nchunk),
        out_type=jax.ShapeDtypeStruct((_NC, n_pad, w), f32),
        mesh=plsc.VectorSubcoreMesh(core_axis_name="c", subcore_axis_name="s"),
        scratch_types=[
            pltpu.VMEM_SHARED((n_pad, w), f32),
            pltpu.VMEM((nchunk, _CH), jnp.int32),
            ib, ib, ib, ib, ib, ib,
            ch_l, ch_l,
            ch_f, ch_f,
            ch_f, ch_f,
            sem, sem, sem, sem,
        ],
        compiler_params=pltpu.CompilerParams(use_tc_tiling_on_sc=False),
    )
    acc_p = sc_edge(fs, erp, packed, z)

    emat = (jnp.arange(kd)[None, :] // d_out ==
            jnp.arange(_L)[:, None]).astype(f32)
    out = pl.pallas_call(
        functools.partial(_combine_body, kd),
        out_shape=jax.ShapeDtypeStruct((n_pad, kd), f32),
    )(acc_p, emat)
    return out[:n]


# speedup vs baseline: 1.7333x; 1.0037x over previous
import functools

import jax
import jax.numpy as jnp
from jax import lax
from jax.experimental import pallas as pl
from jax.experimental.pallas import tpu as pltpu
from jax.experimental.pallas import tpu_sc as plsc

NEG_SLOPE = 0.2
_NC, _NS, _L = 2, 16, 16
_CH = 48


def _proj_body(x_ref, wf_ref, bf_ref, wr_ref, br_ref, f_ref, er_ref):
    x = x_ref[...]
    hi = jax.lax.Precision.HIGHEST
    f_ref[...] = jnp.dot(x, wf_ref[...], precision=hi,
                         preferred_element_type=jnp.float32) + bf_ref[...]
    er_ref[...] = jnp.dot(x, wr_ref[...], precision=hi,
                          preferred_element_type=jnp.float32) + br_ref[...]


def _edge_body(n_pad, k, d_out, nchunk,
               f_hbm, er_hbm, idx_hbm, z_hbm,
               acc_out,
               acc, idx_i,
               sg0, sg1, dg0, dg1, ds0, ds1,
               r_v0, r_v1, f_v0, f_v1, o_v0, o_v1,
               gs0, gs1, ss0, ss1):
    cid = lax.axis_index("c")
    sid = lax.axis_index("s")
    wid = sid * _NC + cid
    kd = k * d_out
    w = kd + _L

    src_g = (sg0, sg1)
    dst_g = (dg0, dg1)
    dst_s = (ds0, ds1)
    r_v = (r_v0, r_v1)
    f_v = (f_v0, f_v1)
    o_v = (o_v0, o_v1)
    gsem = (gs0, gs1)
    ssem = (ss0, ss1)

    rows = n_pad // _NS
    r0 = sid * rows
    pltpu.sync_copy(z_hbm.at[pl.ds(r0, rows)], acc.at[pl.ds(r0, rows)])
    pltpu.sync_copy(idx_hbm.at[pl.ds(wid * nchunk, nchunk)], idx_i)
    plsc.subcore_barrier()

    def unpack(c, sbuf, dbuf):
        for j in range(_CH // _L):
            wv = idx_i[c, pl.ds(j * _L, _L)]
            if sbuf is not None:
                sbuf[pl.ds(j * _L, _L)] = jnp.bitwise_and(wv, 0xFFFF)
            dbuf[pl.ds(j * _L, _L)] = lax.shift_right_logical(wv, 16)

    def issue_gathers(b):
        pltpu.async_copy(f_hbm.at[src_g[b]], f_v[b], gsem[b])
        pltpu.async_copy(er_hbm.at[dst_g[b]], r_v[b], gsem[b])

    def wait_gathers(b):
        pltpu.make_async_copy(f_hbm.at[pl.ds(0, _CH)], f_v[b], gsem[b]).wait()
        pltpu.make_async_copy(er_hbm.at[pl.ds(0, _CH)], r_v[b], gsem[b]).wait()

    def wait_scatter(b):
        pltpu.make_async_copy(f_hbm.at[pl.ds(0, _CH)], o_v[b], ssem[b]).wait()

    def compute(c, b, first):
        wait_gathers(b)

        @pl.when(c + 2 < nchunk)
        def _():
            unpack(c + 2, src_g[b], dst_g[b])

        def ebody(j):
            v = f_v[b][j, pl.ds(kd, _L)] + r_v[b][j, :]
            v = jnp.where(v > 0.0, v, NEG_SLOPE * v)
            sv = jnp.exp(v)
            o_v[b][j, pl.ds(kd, _L)] = sv
            for kk in range(k):
                sl = pl.ds(kk * d_out, d_out)
                o_v[b][j, sl] = f_v[b][j, sl] * sv[kk]

        plsc.parallel_loop(0, _CH, 1, unroll=4)(ebody)
        if not first:
            wait_scatter(b)
        unpack(c, None, dst_s[b])
        pltpu.async_copy(o_v[b], acc.at[dst_s[b]], ssem[b], add=True)

        @pl.when(c + 2 < nchunk)
        def _():
            issue_gathers(b)

    unpack(0, src_g[0], dst_g[0])
    issue_gathers(0)
    unpack(1, src_g[1], dst_g[1])
    issue_gathers(1)
    compute(0, 0, True)
    compute(1, 1, True)

    def pair_body(i, carry):
        compute(2 * i, 0, False)
        compute(2 * i + 1, 1, False)
        return carry

    lax.fori_loop(1, nchunk // 2, pair_body, 0)
    wait_scatter(0)
    wait_scatter(1)
    plsc.subcore_barrier()

    pltpu.sync_copy(acc.at[pl.ds(r0, rows)], acc_out.at[cid, pl.ds(r0, rows)])


def _combine_body(kd, acc_ref, em_ref, o_ref):
    a = acc_ref[0] + acc_ref[1]
    d = a[:, kd:]
    dexp = jnp.dot(d, em_ref[...], precision=jax.lax.Precision.HIGHEST,
                   preferred_element_type=jnp.float32)
    o_ref[...] = jnp.where(dexp > 0.0, a[:, :kd] / dexp, 0.0)


def kernel(feat, edge_index, W_src, b_src, W_dst, b_dst, attn_src):
    n, d_in = feat.shape
    e = edge_index.shape[1]
    k = attn_src.shape[0]
    d_out = attn_src.shape[1] // 2
    kd = k * d_out
    w = kd + _L
    f32 = jnp.float32

    nw = _NC * _NS
    assert d_out == _L and n < 0x8000
    nchunk = (((e + nw * _CH - 1) // (nw * _CH)) + 1) // 2 * 2
    e_pad = nw * nchunk * _CH
    n_pad = ((n + 1 + 8 * _NS - 1) // (8 * _NS)) * (8 * _NS)

    attn_l = attn_src[:, :d_out]
    attn_r = attn_src[:, d_out:]
    w_el = jnp.einsum('dkc,kc->dk', W_src.reshape(d_in, k, d_out), attn_l)
    b_el = jnp.einsum('kc,kc->k', b_src.reshape(k, d_out), attn_l)
    w_er = jnp.einsum('dkc,kc->dk', W_dst.reshape(d_in, k, d_out), attn_r)
    b_er = jnp.einsum('kc,kc->k', b_dst.reshape(k, d_out), attn_r)
    pad = _L - k
    w_f = jnp.concatenate([W_src, jnp.pad(w_el, ((0, 0), (0, pad)))], axis=1)
    b_f = jnp.concatenate([b_src, jnp.pad(b_el, ((0, pad),))])
    w_er_p = jnp.pad(w_er, ((0, 0), (0, pad)))
    b_er_p = jnp.pad(b_er, ((0, pad),))

    fs, erp = pl.pallas_call(
        _proj_body,
        out_shape=(
            jax.ShapeDtypeStruct((n, w), f32),
            jax.ShapeDtypeStruct((n, _L), f32),
        ),
    )(feat, w_f, b_f, w_er_p, b_er_p)

    packed = jnp.concatenate([
        edge_index[0] + (edge_index[1] << 16),
        jnp.full((e_pad - e,), n << 16, jnp.int32),
    ]).reshape(nw * nchunk, _CH)
    z = jnp.zeros((n_pad, w), f32)

    ib = pltpu.VMEM((_CH,), jnp.int32)
    ch_l = pltpu.VMEM((_CH, _L), f32)
    ch_f = pltpu.VMEM((_CH, w), f32)
    sem = pltpu.SemaphoreType.DMA
    sc_edge = pl.kernel(
        functools.partial(_edge_body, n_pad, k, d_out, nchunk),
        out_type=jax.ShapeDtypeStruct((_NC, n_pad, w), f32),
        mesh=plsc.VectorSubcoreMesh(core_axis_name="c", subcore_axis_name="s"),
        scratch_types=[
            pltpu.VMEM_SHARED((n_pad, w), f32),
            pltpu.VMEM((nchunk, _CH), jnp.int32),
            ib, ib, ib, ib, ib, ib,
            ch_l, ch_l,
            ch_f, ch_f,
            ch_f, ch_f,
            sem, sem, sem, sem,
        ],
        compiler_params=pltpu.CompilerParams(use_tc_tiling_on_sc=False),
    )
    acc_p = sc_edge(fs, erp, packed, z)

    emat = (jnp.arange(kd)[None, :] // d_out ==
            jnp.arange(_L)[:, None]).astype(f32)
    out = pl.pallas_call(
        functools.partial(_combine_body, kd),
        out_shape=jax.ShapeDtypeStruct((n_pad, kd), f32),
    )(acc_p, emat)
    return out[:n]
